# baseline (speedup 1.0000x reference)
import functools

import jax
import jax.numpy as jnp
from jax import lax
from jax.experimental import pallas as pl
from jax.experimental.pallas import tpu as pltpu
from jax.experimental.pallas import tpu_sc as plsc

_N = 10000
_NEG = 0.2
_NC = 2
_NS = 16
_L = 16
_NW = _NC * _NS
_G = 48
_E = 330000
_GPW = 216
_EPAD = _NW * _GPW * _G
_EALLOC = _EPAD + 2 * _G
_NPAD = 10016
_RPT = _NPAD // _NS


def _sc_gat_edges(l, r_pad, src, dst, att, C, unroll=4, _G=_G, gpw0=_GPW, gpw1=_GPW):
    CW = C + 16
    NK = C // _L
    mesh = plsc.VectorSubcoreMesh(core_axis_name="c", subcore_axis_name="s")

    def body(l_hbm, r_hbm, src_hbm, dst_hbm, att_hbm, out_hbm,
             s0, s1, s2, s3, d0, d1, d2, d3,
             lrows, rrows, wrows, attv, acc_sh,
             si0, si1, si2, si3, sgl0, sgl1, sgr0, sgr1, ss0, ss1):
        sv = [s0, s1, s2, s3]
        dv = [d0, d1, d2, d3]
        si = [si0, si1, si2, si3]
        sgl = [sgl0, sgl1]
        sgr = [sgr0, sgr1]
        ss = [ss0, ss1]
        ci = lax.axis_index("c")
        ti = lax.axis_index("s")
        gpw_local = jnp.where(ci == 0, gpw0, gpw1)
        gstart = ci * (_NS * gpw0) + ti * gpw_local
        zeros16 = jnp.zeros((_L,), jnp.float32)

        def zrow(i, carry):
            for k in range(CW // _L):
                wrows[0, i, pl.ds(k * _L, _L)] = zeros16
            return carry
        lax.fori_loop(0, _G, zrow, 0)

        row0 = ti * _RPT
        nch = -(-_RPT // _G)
        for j in range(nch):
            off = min(j * _G, _RPT - _G)
            pltpu.sync_copy(wrows.at[0], acc_sh.at[pl.ds(row0 + off, _G)])
        pltpu.sync_copy(att_hbm, attv)
        plsc.subcore_barrier()

        def issue_idx(g, slot):
            base = (gstart + g) * _G
            pltpu.async_copy(src_hbm.at[pl.ds(base, _G)], sv[slot], si[slot])
            pltpu.async_copy(dst_hbm.at[pl.ds(base, _G)], dv[slot], si[slot])

        def wait_idx(slot):
            pltpu.make_async_copy(src_hbm.at[pl.ds(0, _G)], sv[slot], si[slot]).wait()
            pltpu.make_async_copy(dst_hbm.at[pl.ds(0, _G)], dv[slot], si[slot]).wait()

        def issue_gather(b, slot):
            pltpu.async_copy(l_hbm.at[sv[slot]], lrows.at[b], sgl[b])
            pltpu.async_copy(r_hbm.at[dv[slot]], rrows.at[b], sgr[b])

        def wait_gather(b):
            pltpu.make_async_copy(l_hbm.at[pl.ds(0, _G)], lrows.at[b], sgl[b]).wait()
            pltpu.make_async_copy(r_hbm.at[pl.ds(0, _G)], rrows.at[b], sgr[b]).wait()

        def issue_scatter(b, slot):
            pltpu.async_copy(wrows.at[b], acc_sh.at[dv[slot]], ss[b], add=True)

        def wait_scatter(b, slot):
            pltpu.make_async_copy(wrows.at[b], acc_sh.at[dv[slot]], ss[b]).wait()

        def compute(b):
            attk = [attv[pl.ds(k * _L, _L)] for k in range(NK)]

            @plsc.parallel_loop(0, _G, 1, unroll=unroll)
            def _(e):
                acc = None
                lch = []
                for k in range(NK):
                    vl = lrows[b, e, pl.ds(k * _L, _L)]
                    vr = rrows[b, e, pl.ds(k * _L, _L)]
                    lch.append(vl)
                    s = vl + vr
                    z = jnp.maximum(s, 0.0) + _NEG * jnp.minimum(s, 0.0)
                    t = attk[k] * z
                    acc = t if acc is None else acc + t
                w = jnp.exp(jnp.full((_L,), jnp.sum(acc), jnp.float32))
                for k in range(NK):
                    wrows[b, e, pl.ds(k * _L, _L)] = w * lch[k]
                wrows[b, e, pl.ds(C, _L)] = w

        def stage(g, a, skip_scatter_wait=False):
            R = a % 2
            wait_gather(R)
            if not skip_scatter_wait:
                wait_scatter(R, (a + 2) % 4)
            wait_idx((a + 1) % 4)
            issue_gather(1 - R, (a + 1) % 4)
            compute(R)
            issue_scatter(R, a)
            issue_idx(g + 2, (a + 2) % 4)

        issue_idx(0, 0)
        issue_idx(1, 1)
        wait_idx(0)
        issue_gather(0, 0)
        for a in range(4):
            stage(a, a, skip_scatter_wait=(a < 2))

        def quad(q, carry):
            g = q * 4
            for a in range(4):
                stage(g + a, a)
            return carry
        lax.fori_loop(1, gpw_local // 4, quad, 0)

        wait_gather(0)
        wait_scatter(0, 2)
        wait_scatter(1, 3)
        wait_idx(1)

        plsc.subcore_barrier()
        for j in range(nch):
            off = min(j * _G, _RPT - _G)
            sl = pl.ds(row0 + off, _G)
            pltpu.sync_copy(acc_sh.at[sl], out_hbm.at[ci, sl])

    k = pl.kernel(
        body,
        out_type=jax.ShapeDtypeStruct((_NC, _NPAD, CW), jnp.float32),
        mesh=mesh,
        compiler_params=pltpu.CompilerParams(needs_layout_passes=False,
                                             use_tc_tiling_on_sc=False),
        scratch_types=[
            pltpu.VMEM((_G,), jnp.int32),
            pltpu.VMEM((_G,), jnp.int32),
            pltpu.VMEM((_G,), jnp.int32),
            pltpu.VMEM((_G,), jnp.int32),
            pltpu.VMEM((_G,), jnp.int32),
            pltpu.VMEM((_G,), jnp.int32),
            pltpu.VMEM((_G,), jnp.int32),
            pltpu.VMEM((_G,), jnp.int32),
            pltpu.VMEM((2, _G, C), jnp.float32),
            pltpu.VMEM((2, _G, C), jnp.float32),
            pltpu.VMEM((2, _G, CW), jnp.float32),
            pltpu.VMEM((C,), jnp.float32),
            pltpu.VMEM_SHARED((_NPAD, CW), jnp.float32),
            pltpu.SemaphoreType.DMA,
            pltpu.SemaphoreType.DMA,
            pltpu.SemaphoreType.DMA,
            pltpu.SemaphoreType.DMA,
            pltpu.SemaphoreType.DMA,
            pltpu.SemaphoreType.DMA,
            pltpu.SemaphoreType.DMA,
            pltpu.SemaphoreType.DMA,
            pltpu.SemaphoreType.DMA,
            pltpu.SemaphoreType.DMA,
        ],
    )
    return k(l, r_pad, src, dst, att)


def _matmul2(x, Wl, Wr, nb=10):
    N, F = x.shape
    C = Wl.shape[1]

    def body(x_ref, wl_ref, wr_ref, l_ref, r_ref):
        l_ref[...] = x_ref[...] @ wl_ref[...]
        r_ref[...] = x_ref[...] @ wr_ref[...]

    return pl.pallas_call(
        body,
        out_shape=(jax.ShapeDtypeStruct((N, C), x.dtype),
                   jax.ShapeDtypeStruct((N, C), x.dtype)),
        grid=(nb,),
        in_specs=[
            pl.BlockSpec((N // nb, F), lambda i: (i, 0)),
            pl.BlockSpec((F, C), lambda i: (0, 0)),
            pl.BlockSpec((F, C), lambda i: (0, 0)),
        ],
        out_specs=(
            pl.BlockSpec((N // nb, C), lambda i: (i, 0)),
            pl.BlockSpec((N // nb, C), lambda i: (i, 0)),
        ),
    )(x, Wl, Wr)


def _mid(p0, p1_, b, Wl, Wr, nb=4):
    Np, CW = p0.shape
    C = CW - 16
    K = Wl.shape[1]

    def body(p0_ref, p1_ref, b_ref, wl_ref, wr_ref, l_ref, r_ref):
        x = p0_ref[...] + p1_ref[...]
        num = x[:, :C]
        den = jnp.sum(x[:, C:], axis=1, keepdims=True) * (1.0 / 16.0) + 1e-16
        y = num / den + b_ref[...]
        h = jnp.where(y > 0, y, jnp.exp(jnp.minimum(y, 0.0)) - 1.0)
        l_ref[...] = h @ wl_ref[...]
        r_ref[...] = h @ wr_ref[...]

    return pl.pallas_call(
        body,
        out_shape=(jax.ShapeDtypeStruct((Np, K), p0.dtype),
                   jax.ShapeDtypeStruct((Np, K), p0.dtype)),
        grid=(nb,),
        in_specs=[
            pl.BlockSpec((Np // nb, CW), lambda i: (i, 0)),
            pl.BlockSpec((Np // nb, CW), lambda i: (i, 0)),
            pl.BlockSpec((1, C), lambda i: (0, 0)),
            pl.BlockSpec((C, K), lambda i: (0, 0)),
            pl.BlockSpec((C, K), lambda i: (0, 0)),
        ],
        out_specs=(
            pl.BlockSpec((Np // nb, K), lambda i: (i, 0)),
            pl.BlockSpec((Np // nb, K), lambda i: (i, 0)),
        ),
    )(p0, p1_, b, Wl, Wr)


def _final(p0, p1_, b, nb=4):
    Np, CW = p0.shape
    K = CW - 16

    def body(p0_ref, p1_ref, b_ref, o_ref):
        x = p0_ref[...] + p1_ref[...]
        num = x[:, :K]
        den = jnp.sum(x[:, K:], axis=1, keepdims=True) * (1.0 / 16.0) + 1e-16
        y = num / den + b_ref[...]
        m = jnp.max(y, axis=1, keepdims=True)
        ez = jnp.exp(y - m)
        o_ref[...] = ez / jnp.sum(ez, axis=1, keepdims=True)

    return pl.pallas_call(
        body,
        out_shape=jax.ShapeDtypeStruct((Np, K), p0.dtype),
        grid=(nb,),
        in_specs=[
            pl.BlockSpec((Np // nb, CW), lambda i: (i, 0)),
            pl.BlockSpec((Np // nb, CW), lambda i: (i, 0)),
            pl.BlockSpec((1, K), lambda i: (0, 0)),
        ],
        out_specs=pl.BlockSpec((Np // nb, K), lambda i: (i, 0)),
    )(p0, p1_, b)


def kernel(X, ei_feat, batch, Wl1, Wr1, att1, b1, Wl2, Wr2, att2, b2):
    N = X.shape[0]
    loop = jnp.arange(N, dtype=jnp.int32)

    def pad_edges(g, gpw0, gpw1):
        npad_e = _NS * (gpw0 + gpw1) * g + 2 * g - _E
        s = jnp.concatenate([ei_feat[0].astype(jnp.int32), loop,
                             jnp.zeros((npad_e,), jnp.int32)])
        d = jnp.concatenate([ei_feat[1].astype(jnp.int32), loop,
                             N + (jnp.arange(npad_e, dtype=jnp.int32) % 16)])
        return s, d

    src, dst = pad_edges(_G, 184, 248)
    src2, dst2 = pad_edges(128, 64, 104)

    Xp = jnp.concatenate([X, jnp.zeros((_NPAD - N, X.shape[1]), X.dtype)])

    l1, r1 = _matmul2(Xp, Wl1, Wr1, nb=4)
    p1 = _sc_gat_edges(l1, r1, src, dst, att1, 128, unroll=2,
                       gpw0=184, gpw1=248)
    l2, r2 = _mid(p1[0], p1[1], b1.reshape(1, -1), Wl2, Wr2)

    p2 = _sc_gat_edges(l2, r2, src2, dst2, att2, 16, unroll=4, _G=128,
                       gpw0=64, gpw1=104)
    return _final(p2[0], p2[1], b2.reshape(1, -1))[:N]


# speedup vs baseline: 1.1829x; 1.1829x over previous
import functools

import jax
import jax.numpy as jnp
from jax import lax
from jax.experimental import pallas as pl
from jax.experimental.pallas import tpu as pltpu
from jax.experimental.pallas import tpu_sc as plsc

_N = 10000
_NEG = 0.2
_NC = 2
_NS = 16
_L = 16
_NW = _NC * _NS
_G = 48
_E = 330000
_GPW = 216
_EPAD = _NW * _GPW * _G
_EALLOC = _EPAD + 2 * _G
_NPAD = 10016
_RPT = _NPAD // _NS


def _sc_gat_edges(l, r_pad, src, dst, att, C, unroll=4, _G=_G, gpw0=_GPW, gpw1=_GPW):
    CW = C + 16
    NK = C // _L
    mesh = plsc.VectorSubcoreMesh(core_axis_name="c", subcore_axis_name="s")

    def body(l_hbm, r_hbm, src_hbm, dst_hbm, att_hbm, out_hbm,
             s0, s1, s2, s3, d0, d1, d2, d3,
             lrows, rrows, wrows, attv, acc_sh,
             si0, si1, si2, si3, sgl0, sgl1, sgr0, sgr1, ss0, ss1):
        sv = [s0, s1, s2, s3]
        dv = [d0, d1, d2, d3]
        si = [si0, si1, si2, si3]
        sgl = [sgl0, sgl1]
        sgr = [sgr0, sgr1]
        ss = [ss0, ss1]
        ci = lax.axis_index("c")
        ti = lax.axis_index("s")
        gpw_local = jnp.where(ci == 0, gpw0, gpw1)
        gstart = ci * (_NS * gpw0) + ti * gpw_local
        zeros16 = jnp.zeros((_L,), jnp.float32)

        def zrow(i, carry):
            for k in range(CW // _L):
                wrows[0, i, pl.ds(k * _L, _L)] = zeros16
            return carry
        lax.fori_loop(0, _G, zrow, 0)

        row0 = ti * _RPT
        nch = -(-_RPT // _G)
        for j in range(nch):
            off = min(j * _G, _RPT - _G)
            pltpu.sync_copy(wrows.at[0], acc_sh.at[pl.ds(row0 + off, _G)])
        pltpu.sync_copy(att_hbm, attv)
        plsc.subcore_barrier()

        def issue_idx(g, slot):
            base = (gstart + g) * _G
            pltpu.async_copy(src_hbm.at[pl.ds(base, _G)], sv[slot], si[slot])
            pltpu.async_copy(dst_hbm.at[pl.ds(base, _G)], dv[slot], si[slot])

        def wait_idx(slot):
            pltpu.make_async_copy(src_hbm.at[pl.ds(0, _G)], sv[slot], si[slot]).wait()
            pltpu.make_async_copy(dst_hbm.at[pl.ds(0, _G)], dv[slot], si[slot]).wait()

        def issue_gather(b, slot):
            pltpu.async_copy(l_hbm.at[sv[slot]], lrows.at[b], sgl[b])
            pltpu.async_copy(r_hbm.at[dv[slot]], rrows.at[b], sgr[b])

        def wait_gather(b):
            pltpu.make_async_copy(l_hbm.at[pl.ds(0, _G)], lrows.at[b], sgl[b]).wait()
            pltpu.make_async_copy(r_hbm.at[pl.ds(0, _G)], rrows.at[b], sgr[b]).wait()

        def issue_scatter(b, slot):
            pltpu.async_copy(wrows.at[b], acc_sh.at[dv[slot]], ss[b], add=True)

        def wait_scatter(b, slot):
            pltpu.make_async_copy(wrows.at[b], acc_sh.at[dv[slot]], ss[b]).wait()

        def compute(b):
            attk = [attv[pl.ds(k * _L, _L)] for k in range(NK)]

            @plsc.parallel_loop(0, _G, 1, unroll=unroll)
            def _(e):
                acc = None
                lch = []
                for k in range(NK):
                    vl = lrows[b, e, pl.ds(k * _L, _L)]
                    vr = rrows[b, e, pl.ds(k * _L, _L)]
                    lch.append(vl)
                    s = vl + vr
                    z = jnp.maximum(s, 0.0) + _NEG * jnp.minimum(s, 0.0)
                    t = attk[k] * z
                    acc = t if acc is None else acc + t
                w = jnp.exp(jnp.full((_L,), jnp.sum(acc), jnp.float32))
                for k in range(NK):
                    wrows[b, e, pl.ds(k * _L, _L)] = w * lch[k]
                wrows[b, e, pl.ds(C, _L)] = w

        def stage(g, a, skip_scatter_wait=False):
            R = a % 2
            wait_gather(R)
            if not skip_scatter_wait:
                wait_scatter(R, (a + 2) % 4)
            wait_idx((a + 1) % 4)
            issue_gather(1 - R, (a + 1) % 4)
            compute(R)
            issue_scatter(R, a)
            issue_idx(g + 2, (a + 2) % 4)

        issue_idx(0, 0)
        issue_idx(1, 1)
        wait_idx(0)
        issue_gather(0, 0)
        for a in range(4):
            stage(a, a, skip_scatter_wait=(a < 2))

        def quad(q, carry):
            g = q * 4
            for a in range(4):
                stage(g + a, a)
            return carry
        lax.fori_loop(1, gpw_local // 4, quad, 0)

        wait_gather(0)
        wait_scatter(0, 2)
        wait_scatter(1, 3)
        wait_idx(1)

        plsc.subcore_barrier()
        for j in range(nch):
            off = min(j * _G, _RPT - _G)
            sl = pl.ds(row0 + off, _G)
            pltpu.sync_copy(acc_sh.at[sl], out_hbm.at[ci, sl])

    k = pl.kernel(
        body,
        out_type=jax.ShapeDtypeStruct((_NC, _NPAD, CW), jnp.float32),
        mesh=mesh,
        compiler_params=pltpu.CompilerParams(needs_layout_passes=False,
                                             use_tc_tiling_on_sc=False),
        scratch_types=[
            pltpu.VMEM((_G,), jnp.int32),
            pltpu.VMEM((_G,), jnp.int32),
            pltpu.VMEM((_G,), jnp.int32),
            pltpu.VMEM((_G,), jnp.int32),
            pltpu.VMEM((_G,), jnp.int32),
            pltpu.VMEM((_G,), jnp.int32),
            pltpu.VMEM((_G,), jnp.int32),
            pltpu.VMEM((_G,), jnp.int32),
            pltpu.VMEM((2, _G, C), jnp.float32),
            pltpu.VMEM((2, _G, C), jnp.float32),
            pltpu.VMEM((2, _G, CW), jnp.float32),
            pltpu.VMEM((C,), jnp.float32),
            pltpu.VMEM_SHARED((_NPAD, CW), jnp.float32),
            pltpu.SemaphoreType.DMA,
            pltpu.SemaphoreType.DMA,
            pltpu.SemaphoreType.DMA,
            pltpu.SemaphoreType.DMA,
            pltpu.SemaphoreType.DMA,
            pltpu.SemaphoreType.DMA,
            pltpu.SemaphoreType.DMA,
            pltpu.SemaphoreType.DMA,
            pltpu.SemaphoreType.DMA,
            pltpu.SemaphoreType.DMA,
        ],
    )
    return k(l, r_pad, src, dst, att)


def _matmul2(x, Wl, Wr, nb=10):
    N, F = x.shape
    C = Wl.shape[1]

    def body(x_ref, wl_ref, wr_ref, l_ref, r_ref):
        l_ref[...] = x_ref[...] @ wl_ref[...]
        r_ref[...] = x_ref[...] @ wr_ref[...]

    return pl.pallas_call(
        body,
        out_shape=(jax.ShapeDtypeStruct((N, C), x.dtype),
                   jax.ShapeDtypeStruct((N, C), x.dtype)),
        grid=(nb,),
        in_specs=[
            pl.BlockSpec((N // nb, F), lambda i: (i, 0)),
            pl.BlockSpec((F, C), lambda i: (0, 0)),
            pl.BlockSpec((F, C), lambda i: (0, 0)),
        ],
        out_specs=(
            pl.BlockSpec((N // nb, C), lambda i: (i, 0)),
            pl.BlockSpec((N // nb, C), lambda i: (i, 0)),
        ),
    )(x, Wl, Wr)


def _mid(p0, p1_, b, Wl, Wr, nb=4):
    Np, CW = p0.shape
    C = CW - 16
    K = Wl.shape[1]

    def body(p0_ref, p1_ref, b_ref, wl_ref, wr_ref, l_ref, r_ref):
        x = p0_ref[...] + p1_ref[...]
        num = x[:, :C]
        den = jnp.sum(x[:, C:], axis=1, keepdims=True) * (1.0 / 16.0) + 1e-16
        y = num / den + b_ref[...]
        h = jnp.where(y > 0, y, jnp.exp(jnp.minimum(y, 0.0)) - 1.0)
        l_ref[...] = h @ wl_ref[...]
        r_ref[...] = h @ wr_ref[...]

    return pl.pallas_call(
        body,
        out_shape=(jax.ShapeDtypeStruct((Np, K), p0.dtype),
                   jax.ShapeDtypeStruct((Np, K), p0.dtype)),
        grid=(nb,),
        in_specs=[
            pl.BlockSpec((Np // nb, CW), lambda i: (i, 0)),
            pl.BlockSpec((Np // nb, CW), lambda i: (i, 0)),
            pl.BlockSpec((1, C), lambda i: (0, 0)),
            pl.BlockSpec((C, K), lambda i: (0, 0)),
            pl.BlockSpec((C, K), lambda i: (0, 0)),
        ],
        out_specs=(
            pl.BlockSpec((Np // nb, K), lambda i: (i, 0)),
            pl.BlockSpec((Np // nb, K), lambda i: (i, 0)),
        ),
    )(p0, p1_, b, Wl, Wr)


def _final(p0, p1_, b, nb=4):
    Np, CW = p0.shape
    K = CW - 16

    def body(p0_ref, p1_ref, b_ref, o_ref):
        x = p0_ref[...] + p1_ref[...]
        num = x[:, :K]
        den = jnp.sum(x[:, K:], axis=1, keepdims=True) * (1.0 / 16.0) + 1e-16
        y = num / den + b_ref[...]
        m = jnp.max(y, axis=1, keepdims=True)
        ez = jnp.exp(y - m)
        o_ref[...] = ez / jnp.sum(ez, axis=1, keepdims=True)

    return pl.pallas_call(
        body,
        out_shape=jax.ShapeDtypeStruct((Np, K), p0.dtype),
        grid=(nb,),
        in_specs=[
            pl.BlockSpec((Np // nb, CW), lambda i: (i, 0)),
            pl.BlockSpec((Np // nb, CW), lambda i: (i, 0)),
            pl.BlockSpec((1, K), lambda i: (0, 0)),
        ],
        out_specs=pl.BlockSpec((Np // nb, K), lambda i: (i, 0)),
    )(p0, p1_, b)


def kernel(X, ei_feat, batch, Wl1, Wr1, att1, b1, Wl2, Wr2, att2, b2):
    N = X.shape[0]
    loop = jnp.arange(N, dtype=jnp.int32)

    def pad_edges(g, gpw0, gpw1):
        npad_e = _NS * (gpw0 + gpw1) * g + 2 * g - _E
        s = jnp.concatenate([ei_feat[0].astype(jnp.int32), loop,
                             jnp.zeros((npad_e,), jnp.int32)])
        d = jnp.concatenate([ei_feat[1].astype(jnp.int32), loop,
                             N + (jnp.arange(npad_e, dtype=jnp.int32) % 16)])
        return s, d

    src, dst = pad_edges(_G, 248, 184)
    src2, dst2 = pad_edges(128, 104, 64)

    Xp = jnp.concatenate([X, jnp.zeros((_NPAD - N, X.shape[1]), X.dtype)])

    l1, r1 = _matmul2(Xp, Wl1, Wr1, nb=4)
    p1 = _sc_gat_edges(l1, r1, src, dst, att1, 128, unroll=2,
                       gpw0=248, gpw1=184)
    l2, r2 = _mid(p1[0], p1[1], b1.reshape(1, -1), Wl2, Wr2)

    p2 = _sc_gat_edges(l2, r2, src2, dst2, att2, 16, unroll=4, _G=128,
                       gpw0=104, gpw1=64)
    return _final(p2[0], p2[1], b2.reshape(1, -1))[:N]
